# baseline (speedup 1.0000x reference)
import functools

import jax
import jax.numpy as jnp
from jax import lax
from jax.experimental import pallas as pl
from jax.experimental.pallas import tpu as pltpu
from jax.experimental.pallas import tpu_sc as plsc

BATCH = 16384
USER_D = 32
ITEM_D = 10
TEST_D = 10
TAG_D = 12

NUM_CORES = 2
NUM_SUBCORES = 16
NUM_WORKERS = NUM_CORES * NUM_SUBCORES
B_PER_W = BATCH // NUM_WORKERS
CHUNK = 128
NCHUNK = B_PER_W // CHUNK
GROUPS = B_PER_W // 16

ITEM_VROWS = 1000000 * ITEM_D // 16
TEST_VROWS = 1000000 * TEST_D // 16
TAG_VROWS = 1000000 * TAG_D // 16


def _body(u_idx, i_idx, te_idx, ta_idx, user_W, item_V, test_V, tag_V,
          out_hbm,
          u_idx_v, i_idx_v, te_idx_v, ta_idx_v,
          i_p1, i_p2, te_p1, te_p2, ta_p1, ta_p2,
          u_rows, i_lo, i_hi, te_lo, te_hi, ta_lo, ta_hi, out_v, sem):
    wid = lax.axis_index("s") * NUM_CORES + lax.axis_index("c")
    row0 = wid * NCHUNK

    pltpu.sync_copy(u_idx.at[pl.ds(row0, NCHUNK), :], u_idx_v)
    pltpu.sync_copy(i_idx.at[pl.ds(row0, NCHUNK), :], i_idx_v)
    pltpu.sync_copy(te_idx.at[pl.ds(row0, NCHUNK), :], te_idx_v)
    pltpu.sync_copy(ta_idx.at[pl.ds(row0, NCHUNK), :], ta_idx_v)

    base = wid * B_PER_W

    for j in range(NCHUNK):
        for k in range(CHUNK // 16):
            sl = pl.ds(k * 16, 16)
            for idx_v, p1, p2, d, vrows in (
                    (i_idx_v, i_p1, i_p2, ITEM_D, ITEM_VROWS),
                    (te_idx_v, te_p1, te_p2, TEST_D, TEST_VROWS),
                    (ta_idx_v, ta_p1, ta_p2, TAG_D, TAG_VROWS)):
                v = idx_v[j, sl]
                pp = lax.shift_right_logical(v * d, 4)
                p1[j, sl] = pp
                p2[j, sl] = jnp.minimum(pp + 1, vrows - 1)

    copies = []
    for j in range(NCHUNK):
        sl = pl.ds(j * CHUNK, CHUNK)
        copies.append(pltpu.async_copy(user_W.at[u_idx_v.at[j]],
                                       u_rows.at[sl, :], sem))
        copies.append(pltpu.async_copy(item_V.at[i_p1.at[j]],
                                       i_lo.at[sl, :], sem))
        copies.append(pltpu.async_copy(item_V.at[i_p2.at[j]],
                                       i_hi.at[sl, :], sem))
        copies.append(pltpu.async_copy(test_V.at[te_p1.at[j]],
                                       te_lo.at[sl, :], sem))
        copies.append(pltpu.async_copy(test_V.at[te_p2.at[j]],
                                       te_hi.at[sl, :], sem))
        copies.append(pltpu.async_copy(tag_V.at[ta_p1.at[j]],
                                       ta_lo.at[sl, :], sem))
        copies.append(pltpu.async_copy(tag_V.at[ta_p2.at[j]],
                                       ta_hi.at[sl, :], sem))
    for c in copies:
        c.wait()

    def group_body(g, _):
        rows = g * 16 + lax.broadcasted_iota(jnp.int32, (16,), 0)
        gsl = pl.ds(g * 16, 16)
        gj = g // (CHUNK // 16)
        gc = pl.ds((g % (CHUNK // 16)) * 16, 16)
        acc = jnp.zeros((16,), jnp.float32)
        w0 = lax.bitwise_and(i_idx_v[gj, gc] * ITEM_D, 15)
        for d in range(ITEM_D):
            u = plsc.load_gather(u_rows, [rows, jnp.full((16,), d, jnp.int32)])
            w = w0 + d
            wm = lax.bitwise_and(w, 15)
            c = jnp.where(w < 16,
                          plsc.load_gather(i_lo, [rows, wm]),
                          plsc.load_gather(i_hi, [rows, wm]))
            acc = acc + u * c
        w0 = lax.bitwise_and(te_idx_v[gj, gc] * TEST_D, 15)
        for d in range(TEST_D):
            u = plsc.load_gather(
                u_rows, [rows, jnp.full((16,), ITEM_D + d, jnp.int32)])
            w = w0 + d
            wm = lax.bitwise_and(w, 15)
            c = jnp.where(w < 16,
                          plsc.load_gather(te_lo, [rows, wm]),
                          plsc.load_gather(te_hi, [rows, wm]))
            acc = acc + u * c
        w0 = lax.bitwise_and(ta_idx_v[gj, gc] * TAG_D, 15)
        for d in range(TAG_D):
            u = plsc.load_gather(
                u_rows,
                [rows, jnp.full((16,), ITEM_D + TEST_D + d, jnp.int32)])
            w = w0 + d
            wm = lax.bitwise_and(w, 15)
            c = jnp.where(w < 16,
                          plsc.load_gather(ta_lo, [rows, wm]),
                          plsc.load_gather(ta_hi, [rows, wm]))
            acc = acc + u * c
        res = 1.0 / (1.0 + jnp.exp(-acc))
        out_v[gsl] = res
        return ()

    lax.fori_loop(0, GROUPS, group_body, (), unroll=False)

    pltpu.sync_copy(out_v, out_hbm.at[pl.ds(base, B_PER_W)])


@functools.partial(jax.jit, static_argnames=("interpret",))
def _run(u_idx, i_idx, te_idx, ta_idx, user_W, item_V, test_V, tag_V,
         interpret=False):
    mesh = plsc.VectorSubcoreMesh(core_axis_name="c", subcore_axis_name="s",
                                  num_cores=NUM_CORES,
                                  num_subcores=NUM_SUBCORES)
    idx2 = pltpu.VMEM((NCHUNK, CHUNK), jnp.int32)
    return pl.kernel(
        _body,
        out_type=jax.ShapeDtypeStruct((BATCH,), jnp.float32),
        mesh=mesh,
        scratch_types=[
            idx2, idx2, idx2, idx2,
            idx2, idx2, idx2, idx2, idx2, idx2,
            pltpu.VMEM((B_PER_W, USER_D), jnp.float32),
            pltpu.VMEM((B_PER_W, 16), jnp.float32),
            pltpu.VMEM((B_PER_W, 16), jnp.float32),
            pltpu.VMEM((B_PER_W, 16), jnp.float32),
            pltpu.VMEM((B_PER_W, 16), jnp.float32),
            pltpu.VMEM((B_PER_W, 16), jnp.float32),
            pltpu.VMEM((B_PER_W, 16), jnp.float32),
            pltpu.VMEM((B_PER_W,), jnp.float32),
            pltpu.SemaphoreType.DMA,
        ],
        compiler_params=pltpu.CompilerParams(
            use_tc_tiling_on_sc=False,
            needs_layout_passes=False,
        ),
        interpret=interpret,
    )(u_idx, i_idx, te_idx, ta_idx, user_W, item_V, test_V, tag_V)


def kernel(data, user_W, item_W, test_W, tag_W):
    u_idx = data[:, 0].reshape(BATCH // CHUNK, CHUNK)
    i_idx = data[:, 1].reshape(BATCH // CHUNK, CHUNK)
    te_idx = data[:, 2].reshape(BATCH // CHUNK, CHUNK)
    ta_idx = data[:, 3].reshape(BATCH // CHUNK, CHUNK)
    item_V = item_W.reshape(ITEM_VROWS, 16)
    test_V = test_W.reshape(TEST_VROWS, 16)
    tag_V = tag_W.reshape(TAG_VROWS, 16)
    return _run(u_idx, i_idx, te_idx, ta_idx, user_W, item_V, test_V, tag_V)


# speedup vs baseline: 1.7090x; 1.7090x over previous
import functools

import jax
import jax.numpy as jnp
from jax import lax
from jax.experimental import pallas as pl
from jax.experimental.pallas import tpu as pltpu
from jax.experimental.pallas import tpu_sc as plsc

BATCH = 16384
USER_D = 32
ITEM_D = 10
TEST_D = 10
TAG_D = 12

NUM_CORES = 2
NUM_SUBCORES = 16
NUM_WORKERS = NUM_CORES * NUM_SUBCORES
B_PER_W = BATCH // NUM_WORKERS
CHUNK = 128
GROUPS = B_PER_W // 16


def _body(u_idx, i_idx, te_idx, ta_idx, user_W, item_W, test_W, tag_W,
          out_hbm, u_idx_v, i_idx_v, te_idx_v, ta_idx_v,
          u_rows, i_rows, te_rows, ta_rows, out_v, sem):
    wid = lax.axis_index("s") * NUM_CORES + lax.axis_index("c")
    base = wid * B_PER_W

    pltpu.sync_copy(u_idx.at[pl.ds(base, B_PER_W)], u_idx_v)
    pltpu.sync_copy(i_idx.at[pl.ds(base, B_PER_W)], i_idx_v)
    pltpu.sync_copy(te_idx.at[pl.ds(base, B_PER_W)], te_idx_v)
    pltpu.sync_copy(ta_idx.at[pl.ds(base, B_PER_W)], ta_idx_v)

    def chunk_body(ch, _):
        ch0 = ch * CHUNK

        def fetch_body(g, _):
            g16 = ch0 + g * 16
            uvec = u_idx_v[pl.ds(g16, 16)]
            ivec = i_idx_v[pl.ds(g16, 16)]
            tevec = te_idx_v[pl.ds(g16, 16)]
            tavec = ta_idx_v[pl.ds(g16, 16)]
            for l in range(16):
                rr = g * 16 + l
                pltpu.make_async_copy(user_W.at[pl.ds(uvec[l], 1), :],
                                      u_rows.at[pl.ds(rr, 1), :],
                                      sem).start()
                pltpu.make_async_copy(item_W.at[pl.ds(ivec[l], 1), :],
                                      i_rows.at[pl.ds(rr, 1), :],
                                      sem).start()
                pltpu.make_async_copy(test_W.at[pl.ds(tevec[l], 1), :],
                                      te_rows.at[pl.ds(rr, 1), :],
                                      sem).start()
                pltpu.make_async_copy(tag_W.at[pl.ds(tavec[l], 1), :],
                                      ta_rows.at[pl.ds(rr, 1), :],
                                      sem).start()
            return ()

        lax.fori_loop(0, CHUNK // 16, fetch_body, (), unroll=False)

        def drain_body(rr, _):
            pltpu.make_async_copy(user_W.at[pl.ds(0, 1), :],
                                  u_rows.at[pl.ds(rr, 1), :], sem).wait()
            pltpu.make_async_copy(item_W.at[pl.ds(0, 1), :],
                                  i_rows.at[pl.ds(rr, 1), :], sem).wait()
            pltpu.make_async_copy(test_W.at[pl.ds(0, 1), :],
                                  te_rows.at[pl.ds(rr, 1), :], sem).wait()
            pltpu.make_async_copy(tag_W.at[pl.ds(0, 1), :],
                                  ta_rows.at[pl.ds(rr, 1), :], sem).wait()
            return ()

        lax.fori_loop(0, CHUNK, drain_body, (), unroll=False)

        def group_body(g, _):
            rows = g * 16 + lax.broadcasted_iota(jnp.int32, (16,), 0)
            acc = jnp.zeros((16,), jnp.float32)
            for d in range(USER_D):
                dvec = jnp.full((16,), d, jnp.int32)
                u = plsc.load_gather(u_rows, [rows, dvec])
                if d < ITEM_D:
                    c = plsc.load_gather(i_rows, [rows, dvec])
                elif d < ITEM_D + TEST_D:
                    c = plsc.load_gather(
                        te_rows,
                        [rows, jnp.full((16,), d - ITEM_D, jnp.int32)])
                else:
                    c = plsc.load_gather(
                        ta_rows,
                        [rows, jnp.full((16,), d - ITEM_D - TEST_D,
                                        jnp.int32)])
                acc = acc + u * c
            res = 1.0 / (1.0 + jnp.exp(-acc))
            out_v[pl.ds(ch0 + g * 16, 16)] = res
            return ()

        lax.fori_loop(0, CHUNK // 16, group_body, (), unroll=False)
        return ()

    lax.fori_loop(0, B_PER_W // CHUNK, chunk_body, (), unroll=False)

    pltpu.sync_copy(out_v, out_hbm.at[pl.ds(base, B_PER_W)])


@functools.partial(jax.jit, static_argnames=("interpret",))
def _run(u_idx, i_idx, te_idx, ta_idx, user_W, item_W, test_W, tag_W,
         interpret=False):
    mesh = plsc.VectorSubcoreMesh(core_axis_name="c", subcore_axis_name="s",
                                  num_cores=NUM_CORES,
                                  num_subcores=NUM_SUBCORES)
    idx1 = pltpu.VMEM((B_PER_W,), jnp.int32)
    return pl.kernel(
        _body,
        out_type=jax.ShapeDtypeStruct((BATCH,), jnp.float32),
        mesh=mesh,
        scratch_types=[
            idx1, idx1, idx1, idx1,
            pltpu.VMEM((CHUNK, USER_D), jnp.float32),
            pltpu.VMEM((CHUNK, ITEM_D), jnp.float32),
            pltpu.VMEM((CHUNK, TEST_D), jnp.float32),
            pltpu.VMEM((CHUNK, TAG_D), jnp.float32),
            pltpu.VMEM((B_PER_W,), jnp.float32),
            pltpu.SemaphoreType.DMA,
        ],
        compiler_params=pltpu.CompilerParams(
            use_tc_tiling_on_sc=True,
            needs_layout_passes=False,
        ),
        interpret=interpret,
    )(u_idx, i_idx, te_idx, ta_idx, user_W, item_W, test_W, tag_W)


def kernel(data, user_W, item_W, test_W, tag_W):
    u_idx = data[:, 0]
    i_idx = data[:, 1]
    te_idx = data[:, 2]
    ta_idx = data[:, 3]
    return _run(u_idx, i_idx, te_idx, ta_idx, user_W, item_W, test_W, tag_W)


# speedup vs baseline: 1.7123x; 1.0019x over previous
import functools

import jax
import jax.numpy as jnp
from jax import lax
from jax.experimental import pallas as pl
from jax.experimental.pallas import tpu as pltpu
from jax.experimental.pallas import tpu_sc as plsc

BATCH = 16384
USER_D = 32
ITEM_D = 10
TEST_D = 10
TAG_D = 12

NUM_CORES = 2
NUM_SUBCORES = 16
NUM_WORKERS = NUM_CORES * NUM_SUBCORES
B_PER_W = BATCH // NUM_WORKERS
CHUNK = 128
GROUPS = B_PER_W // 16


def _body(u_idx, i_idx, te_idx, ta_idx, user_W, item_W, test_W, tag_W,
          out_hbm, u_idx_v, i_idx_v, te_idx_v, ta_idx_v,
          u_rows, i_rows, te_rows, ta_rows, out_v,
          sem_u, sem_i, sem_te, sem_ta):
    wid = lax.axis_index("s") * NUM_CORES + lax.axis_index("c")
    base = wid * B_PER_W

    pltpu.sync_copy(u_idx.at[pl.ds(base, B_PER_W)], u_idx_v)
    pltpu.sync_copy(i_idx.at[pl.ds(base, B_PER_W)], i_idx_v)
    pltpu.sync_copy(te_idx.at[pl.ds(base, B_PER_W)], te_idx_v)
    pltpu.sync_copy(ta_idx.at[pl.ds(base, B_PER_W)], ta_idx_v)

    def chunk_body(ch, _):
        ch0 = ch * CHUNK

        def fetch_body(g, _):
            g16 = ch0 + g * 16
            uvec = u_idx_v[pl.ds(g16, 16)]
            ivec = i_idx_v[pl.ds(g16, 16)]
            tevec = te_idx_v[pl.ds(g16, 16)]
            tavec = ta_idx_v[pl.ds(g16, 16)]
            for l in range(16):
                rr = g * 16 + l
                pltpu.make_async_copy(user_W.at[pl.ds(uvec[l], 1), :],
                                      u_rows.at[pl.ds(rr, 1), :],
                                      sem_u).start()
                pltpu.make_async_copy(item_W.at[pl.ds(ivec[l], 1), :],
                                      i_rows.at[pl.ds(rr, 1), :],
                                      sem_i).start()
                pltpu.make_async_copy(test_W.at[pl.ds(tevec[l], 1), :],
                                      te_rows.at[pl.ds(rr, 1), :],
                                      sem_te).start()
                pltpu.make_async_copy(tag_W.at[pl.ds(tavec[l], 1), :],
                                      ta_rows.at[pl.ds(rr, 1), :],
                                      sem_ta).start()
            return ()

        lax.fori_loop(0, CHUNK // 16, fetch_body, (), unroll=False)

        pltpu.make_async_copy(user_W.at[pl.ds(0, CHUNK), :], u_rows,
                              sem_u).wait()
        pltpu.make_async_copy(item_W.at[pl.ds(0, CHUNK), :], i_rows,
                              sem_i).wait()
        pltpu.make_async_copy(test_W.at[pl.ds(0, CHUNK), :], te_rows,
                              sem_te).wait()
        pltpu.make_async_copy(tag_W.at[pl.ds(0, CHUNK), :], ta_rows,
                              sem_ta).wait()

        def group_body(g, _):
            rows = g * 16 + lax.broadcasted_iota(jnp.int32, (16,), 0)
            acc = jnp.zeros((16,), jnp.float32)
            for d in range(USER_D):
                dvec = jnp.full((16,), d, jnp.int32)
                u = plsc.load_gather(u_rows, [rows, dvec])
                if d < ITEM_D:
                    c = plsc.load_gather(i_rows, [rows, dvec])
                elif d < ITEM_D + TEST_D:
                    c = plsc.load_gather(
                        te_rows,
                        [rows, jnp.full((16,), d - ITEM_D, jnp.int32)])
                else:
                    c = plsc.load_gather(
                        ta_rows,
                        [rows, jnp.full((16,), d - ITEM_D - TEST_D,
                                        jnp.int32)])
                acc = acc + u * c
            res = 1.0 / (1.0 + jnp.exp(-acc))
            out_v[pl.ds(ch0 + g * 16, 16)] = res
            return ()

        lax.fori_loop(0, CHUNK // 16, group_body, (), unroll=False)
        return ()

    lax.fori_loop(0, B_PER_W // CHUNK, chunk_body, (), unroll=False)

    pltpu.sync_copy(out_v, out_hbm.at[pl.ds(base, B_PER_W)])


@functools.partial(jax.jit, static_argnames=("interpret",))
def _run(u_idx, i_idx, te_idx, ta_idx, user_W, item_W, test_W, tag_W,
         interpret=False):
    mesh = plsc.VectorSubcoreMesh(core_axis_name="c", subcore_axis_name="s",
                                  num_cores=NUM_CORES,
                                  num_subcores=NUM_SUBCORES)
    idx1 = pltpu.VMEM((B_PER_W,), jnp.int32)
    return pl.kernel(
        _body,
        out_type=jax.ShapeDtypeStruct((BATCH,), jnp.float32),
        mesh=mesh,
        scratch_types=[
            idx1, idx1, idx1, idx1,
            pltpu.VMEM((CHUNK, USER_D), jnp.float32),
            pltpu.VMEM((CHUNK, ITEM_D), jnp.float32),
            pltpu.VMEM((CHUNK, TEST_D), jnp.float32),
            pltpu.VMEM((CHUNK, TAG_D), jnp.float32),
            pltpu.VMEM((B_PER_W,), jnp.float32),
            pltpu.SemaphoreType.DMA,
            pltpu.SemaphoreType.DMA,
            pltpu.SemaphoreType.DMA,
            pltpu.SemaphoreType.DMA,
        ],
        compiler_params=pltpu.CompilerParams(
            use_tc_tiling_on_sc=True,
            needs_layout_passes=False,
        ),
        interpret=interpret,
    )(u_idx, i_idx, te_idx, ta_idx, user_W, item_W, test_W, tag_W)


def kernel(data, user_W, item_W, test_W, tag_W):
    u_idx = data[:, 0]
    i_idx = data[:, 1]
    te_idx = data[:, 2]
    ta_idx = data[:, 3]
    return _run(u_idx, i_idx, te_idx, ta_idx, user_W, item_W, test_W, tag_W)
